# baseline (speedup 1.0000x reference)
import functools

import jax
import jax.numpy as jnp
from jax import lax
from jax.experimental import pallas as pl
from jax.experimental.pallas import tpu as pltpu
from jax.experimental.pallas import tpu_sc as plsc

N = 10000
D = 128
DH = D // 2
E = 320000

NC = 2
NS = 16
NW = NC * NS
CH = 128
N_PAD = 10112
TRASH = N
NB = 2
E_PAD = ((E + NW * CH * NB - 1) // (NW * CH * NB)) * (NW * CH * NB)
NCH_P = E_PAD // NW // CH
NCH_A = E_PAD // NS // CH
RPT = N_PAD // NS

_MESH = dict(core_axis_name="c", subcore_axis_name="s")


def _zero16():
    return jnp.zeros((16,), jnp.float32)


def _sc_prep():
    out_type = [
        jax.ShapeDtypeStruct((NW, NCH_P, CH), jnp.int32),
        jax.ShapeDtypeStruct((N_PAD,), jnp.float32),
        jax.ShapeDtypeStruct((N_PAD,), jnp.float32),
    ]
    scratch = [
        pltpu.VMEM((NCH_P, CH), jnp.int32),
        pltpu.VMEM((NCH_P, CH), jnp.int32),
        pltpu.VMEM((RPT,), jnp.float32),
        pltpu.VMEM((CH,), jnp.float32),
        pltpu.VMEM_SHARED((N_PAD,), jnp.float32),
    ]

    def body(row3, col3, colp3, d0_out, d1_out, rowv, colv, degv, onesv, deg_sh):
        c = lax.axis_index("c")
        s = lax.axis_index("s")
        wid = c * NS + s

        def zdeg(r, _):
            degv[pl.ds(r * 16, 16)] = _zero16()
            return 0

        lax.fori_loop(0, RPT // 16, zdeg, 0)
        for k in range(CH // 16):
            onesv[pl.ds(k * 16, 16)] = jnp.ones((16,), jnp.float32)
        pltpu.sync_copy(degv, deg_sh.at[pl.ds(s * RPT, RPT)])

        pltpu.sync_copy(row3.at[wid], rowv)
        pltpu.sync_copy(col3.at[wid], colv)

        def fix(i, _):
            for k in range(CH // 16):
                rv = rowv[i, pl.ds(k * 16, 16)]
                cv = colv[i, pl.ds(k * 16, 16)]
                colv[i, pl.ds(k * 16, 16)] = jnp.where(
                    rv == cv, jnp.full((16,), TRASH, jnp.int32), cv)
            return 0

        lax.fori_loop(0, NCH_P, fix, 0)
        pltpu.sync_copy(colv, colp3.at[wid])

        plsc.subcore_barrier()

        def step(i, _):
            pltpu.sync_copy(onesv, deg_sh.at[colv.at[i]], add=True)
            return 0

        lax.fori_loop(0, NCH_P, step, 0)

        plsc.subcore_barrier()

        pltpu.sync_copy(deg_sh.at[pl.ds(s * RPT, RPT)], degv)

        @pl.when(c == 0)
        def _():
            pltpu.sync_copy(degv, d0_out.at[pl.ds(s * RPT, RPT)])

        @pl.when(c == 1)
        def _():
            pltpu.sync_copy(degv, d1_out.at[pl.ds(s * RPT, RPT)])

    return pl.kernel(body, out_type=out_type,
                     mesh=plsc.VectorSubcoreMesh(**_MESH),
                     compiler_params=pltpu.CompilerParams(
                         use_tc_tiling_on_sc=False),
                     scratch_types=scratch)


def _sc_aggregate():
    out_type = [
        jax.ShapeDtypeStruct((N_PAD, DH), jnp.float32),
        jax.ShapeDtypeStruct((N_PAD, DH), jnp.float32),
    ]
    scratch = (
        [pltpu.VMEM((NCH_A, CH), jnp.int32),
         pltpu.VMEM((NCH_A, CH), jnp.int32)]
        + [pltpu.VMEM((CH, DH), jnp.float32)] * NB
        + [pltpu.VMEM((RPT // 2, DH), jnp.float32),
           pltpu.VMEM_SHARED((N_PAD, DH), jnp.float32)]
        + [pltpu.SemaphoreType.DMA] * (2 * NB)
    )

    def body(row3, colp3, xa, xb, s0_out, s1_out, rowv, colv, *rest):
        gbufs = rest[:NB]
        zbuf, s_sh = rest[NB:NB + 2]
        gsem = rest[NB + 2:NB + 2 + NB]
        ssem = rest[NB + 2 + NB:]
        c = lax.axis_index("c")
        s = lax.axis_index("s")
        half = RPT // 2

        def zrow(r, _):
            for k in range(DH // 16):
                zbuf[r, pl.ds(k * 16, 16)] = _zero16()
            return 0

        lax.fori_loop(0, half, zrow, 0)
        pltpu.sync_copy(zbuf, s_sh.at[pl.ds(s * RPT, half)])
        pltpu.sync_copy(zbuf, s_sh.at[pl.ds(s * RPT + half, half)])

        pltpu.sync_copy(row3.at[s], rowv)
        pltpu.sync_copy(colp3.at[s], colv)

        plsc.subcore_barrier()

        def gather(j, b):
            @pl.when(c == 0)
            def _():
                pltpu.async_copy(xa.at[rowv.at[j]], gbufs[b], gsem[b])

            @pl.when(c == 1)
            def _():
                pltpu.async_copy(xb.at[rowv.at[j]], gbufs[b], gsem[b])

        def drain(sem_ref, b):
            pltpu.make_async_copy(xa.at[pl.ds(0, CH)], gbufs[b], sem_ref).wait()

        def phase(i, b):
            drain(gsem[b], b)
            pltpu.async_copy(gbufs[b], s_sh.at[colv.at[i]], ssem[b], add=True)
            j = i + NB - 1
            bj = (b + NB - 1) % NB

            @pl.when(j < NCH_A)
            def _():
                @pl.when(i >= 1)
                def _():
                    drain(ssem[bj], bj)
                gather(j, bj)

        for b in range(NB - 1):
            gather(b, b)

        def group(g, _):
            for b in range(NB):
                phase(g * NB + b, b)
            return 0

        lax.fori_loop(0, NCH_A // NB, group, 0)

        for b in range(NB):
            drain(ssem[b], b)

        plsc.subcore_barrier()

        for h in range(2):
            pltpu.sync_copy(s_sh.at[pl.ds(s * RPT + h * half, half)], zbuf)

            @pl.when(c == 0)
            def _():
                pltpu.sync_copy(zbuf, s0_out.at[pl.ds(s * RPT + h * half, half)])

            @pl.when(c == 1)
            def _():
                pltpu.sync_copy(zbuf, s1_out.at[pl.ds(s * RPT + h * half, half)])

    return pl.kernel(body, out_type=out_type,
                     mesh=plsc.VectorSubcoreMesh(**_MESH),
                     compiler_params=pltpu.CompilerParams(
                         use_tc_tiling_on_sc=False),
                     scratch_types=scratch)


BR = 1264


def _tc_dense_body(first_layer, last_layer, s0, s1, xa, xb, dinv_a, dinv_b,
                   wo_t, wr_t, b, *outs):
    if first_layer:
        dinv = 1.0 / (dinv_a[...] + dinv_b[...] + 1.0)
    else:
        dinv = dinv_a[...]
    x = jnp.concatenate([xa[...], xb[...]], axis=1)
    agg = jnp.concatenate([s0[...] + xa[...], s1[...] + xb[...]], axis=1)
    agg = agg * dinv[:, None]
    out = (jnp.dot(agg, wo_t[...], preferred_element_type=jnp.float32)
           + jnp.dot(x, wr_t[...], preferred_element_type=jnp.float32)
           + b[...][None, :])
    if last_layer:
        outs[0][...] = out
    else:
        out = jnp.maximum(out, 0.0)
        outs[0][...] = out[:, :DH]
        outs[1][...] = out[:, DH:]
    if first_layer:
        outs[2][...] = dinv


def _tc_dense(first_layer: bool, last_layer: bool):
    if last_layer:
        out_shape = [jax.ShapeDtypeStruct((N_PAD, D), jnp.float32)]
    else:
        out_shape = [jax.ShapeDtypeStruct((N_PAD, DH), jnp.float32),
                     jax.ShapeDtypeStruct((N_PAD, DH), jnp.float32)]
    if first_layer:
        out_shape.append(jax.ShapeDtypeStruct((N_PAD,), jnp.float32))
    return pl.pallas_call(
        functools.partial(_tc_dense_body, first_layer, last_layer),
        out_shape=out_shape,
    )


def kernel(x, edge_index, W_out1, b_out1, W_root1, W_out2, b_out2, W_root2,
           W_out3, b_out3, W_root3):
    row = edge_index[0]
    col = edge_index[1]
    pad = E_PAD - E
    row_p = jnp.concatenate([row, jnp.zeros((pad,), jnp.int32)])
    col_p = jnp.concatenate([col, jnp.full((pad,), TRASH, jnp.int32)])
    row3p = row_p.reshape(NW, NCH_P, CH)
    col3p = col_p.reshape(NW, NCH_P, CH)
    row3a = row_p.reshape(NS, NCH_A, CH)
    x_pad = jnp.zeros((N_PAD, D), jnp.float32).at[:N].set(x)
    xa = x_pad[:, :DH]
    xb = x_pad[:, DH:]

    colp3, d0, d1 = _sc_prep()(row3p, col3p)
    colp3a = colp3.reshape(NS, NCH_A, CH)
    sc = _sc_aggregate()

    s0, s1 = sc(row3a, colp3a, xa, xb)
    ha, hb, dinv = _tc_dense(True, False)(
        s0, s1, xa, xb, d0, d1, W_out1.T, W_root1.T, b_out1)

    s0, s1 = sc(row3a, colp3a, ha, hb)
    ha, hb = _tc_dense(False, False)(
        s0, s1, ha, hb, dinv, dinv, W_out2.T, W_root2.T, b_out2)

    s0, s1 = sc(row3a, colp3a, ha, hb)
    out, = _tc_dense(False, True)(
        s0, s1, ha, hb, dinv, dinv, W_out3.T, W_root3.T, b_out3)

    return out[:N]


# speedup vs baseline: 1.6966x; 1.6966x over previous
import functools

import jax
import jax.numpy as jnp
from jax import lax
from jax.experimental import pallas as pl
from jax.experimental.pallas import tpu as pltpu
from jax.experimental.pallas import tpu_sc as plsc

N = 10000
D = 128
DH = D // 2
E = 320000

NC = 2
NS = 16
NW = NC * NS
CH = 128
N_PAD = 10112
TRASH = N
NB = 1
E_PAD = ((E + NW * CH * NB - 1) // (NW * CH * NB)) * (NW * CH * NB)
NCH_P = E_PAD // NW // CH
NCH_A = E_PAD // NS // CH
RPT = N_PAD // NS

_MESH = dict(core_axis_name="c", subcore_axis_name="s")


def _zero16():
    return jnp.zeros((16,), jnp.float32)


def _sc_prep():
    out_type = [
        jax.ShapeDtypeStruct((NW, NCH_P, CH), jnp.int32),
        jax.ShapeDtypeStruct((N_PAD,), jnp.float32),
        jax.ShapeDtypeStruct((N_PAD,), jnp.float32),
    ]
    scratch = [
        pltpu.VMEM((NCH_P, CH), jnp.int32),
        pltpu.VMEM((NCH_P, CH), jnp.int32),
        pltpu.VMEM((RPT,), jnp.float32),
        pltpu.VMEM((CH,), jnp.float32),
        pltpu.VMEM_SHARED((N_PAD,), jnp.float32),
    ]

    def body(row3, col3, colp3, d0_out, d1_out, rowv, colv, degv, onesv, deg_sh):
        c = lax.axis_index("c")
        s = lax.axis_index("s")
        wid = c * NS + s

        def zdeg(r, _):
            degv[pl.ds(r * 16, 16)] = _zero16()
            return 0

        lax.fori_loop(0, RPT // 16, zdeg, 0)
        for k in range(CH // 16):
            onesv[pl.ds(k * 16, 16)] = jnp.ones((16,), jnp.float32)
        pltpu.sync_copy(degv, deg_sh.at[pl.ds(s * RPT, RPT)])

        pltpu.sync_copy(row3.at[wid], rowv)
        pltpu.sync_copy(col3.at[wid], colv)

        def fix(i, _):
            for k in range(CH // 16):
                rv = rowv[i, pl.ds(k * 16, 16)]
                cv = colv[i, pl.ds(k * 16, 16)]
                colv[i, pl.ds(k * 16, 16)] = jnp.where(
                    rv == cv, jnp.full((16,), TRASH, jnp.int32), cv)
            return 0

        lax.fori_loop(0, NCH_P, fix, 0)
        pltpu.sync_copy(colv, colp3.at[wid])

        plsc.subcore_barrier()

        def step(i, _):
            pltpu.sync_copy(onesv, deg_sh.at[colv.at[i]], add=True)
            return 0

        lax.fori_loop(0, NCH_P, step, 0)

        plsc.subcore_barrier()

        pltpu.sync_copy(deg_sh.at[pl.ds(s * RPT, RPT)], degv)

        @pl.when(c == 0)
        def _():
            pltpu.sync_copy(degv, d0_out.at[pl.ds(s * RPT, RPT)])

        @pl.when(c == 1)
        def _():
            pltpu.sync_copy(degv, d1_out.at[pl.ds(s * RPT, RPT)])

    return pl.kernel(body, out_type=out_type,
                     mesh=plsc.VectorSubcoreMesh(**_MESH),
                     compiler_params=pltpu.CompilerParams(
                         use_tc_tiling_on_sc=False),
                     scratch_types=scratch)


def _sc_aggregate():
    out_type = [
        jax.ShapeDtypeStruct((N_PAD, DH), jnp.float32),
        jax.ShapeDtypeStruct((N_PAD, DH), jnp.float32),
    ]
    qr = RPT // 4
    scratch = (
        [pltpu.VMEM((NCH_A, CH), jnp.int32),
         pltpu.VMEM((NCH_A, CH), jnp.int32)]
        + [pltpu.VMEM((2 * NB, CH, DH), jnp.float32),
           pltpu.VMEM((qr, DH), jnp.float32),
           pltpu.VMEM_SHARED((N_PAD, DH), jnp.float32)]
        + [pltpu.SemaphoreType.DMA] * 2
    )

    def body(row3, colp3, xa, xb, s0_out, s1_out, rowv, colv, bbuf,
             zbuf, s_sh, gs0, gs1):
        gsem = (gs0, gs1)
        c = lax.axis_index("c")
        s = lax.axis_index("s")
        nsup = NCH_A // NB

        def zrow(r, _):
            for k in range(DH // 16):
                zbuf[r, pl.ds(k * 16, 16)] = _zero16()
            return 0

        lax.fori_loop(0, qr, zrow, 0)
        for q in range(4):
            pltpu.sync_copy(zbuf, s_sh.at[pl.ds(s * RPT + q * qr, qr)])

        pltpu.sync_copy(row3.at[s], rowv)
        pltpu.sync_copy(colp3.at[s], colv)

        plsc.subcore_barrier()

        def fire_gathers(t, p, sem):
            def fg(k, _):
                dst = bbuf.at[p * NB + k]

                @pl.when(c == 0)
                def _():
                    pltpu.async_copy(xa.at[rowv.at[t * NB + k]], dst, sem)

                @pl.when(c == 1)
                def _():
                    pltpu.async_copy(xb.at[rowv.at[t * NB + k]], dst, sem)

                return 0

            lax.fori_loop(0, NB, fg, 0)

        def scatter_sync(t, p):
            def fs(k, _):
                pltpu.sync_copy(bbuf.at[p * NB + k],
                                s_sh.at[colv.at[t * NB + k]], add=True)
                return 0

            lax.fori_loop(0, NB, fs, 0)

        def drain_big(sem):
            def dr(k, _):
                pltpu.make_async_copy(xa.at[pl.ds(0, CH)], bbuf.at[0], sem).wait()
                return 0

            lax.fori_loop(0, NB, dr, 0)

        def phase(t, p, sp, sq):
            @pl.when(t + 1 < nsup)
            def _():
                fire_gathers(t + 1, 1 - p, sq)

            drain_big(sp)
            scatter_sync(t, p)

        fire_gathers(0, 0, gs0)

        def group(g, _):
            phase(2 * g, 0, gs0, gs1)
            phase(2 * g + 1, 1, gs1, gs0)
            return 0

        lax.fori_loop(0, nsup // 2, group, 0)

        plsc.subcore_barrier()

        for q in range(4):
            pltpu.sync_copy(s_sh.at[pl.ds(s * RPT + q * qr, qr)], zbuf)

            @pl.when(c == 0)
            def _():
                pltpu.sync_copy(zbuf, s0_out.at[pl.ds(s * RPT + q * qr, qr)])

            @pl.when(c == 1)
            def _():
                pltpu.sync_copy(zbuf, s1_out.at[pl.ds(s * RPT + q * qr, qr)])

    return pl.kernel(body, out_type=out_type,
                     mesh=plsc.VectorSubcoreMesh(**_MESH),
                     compiler_params=pltpu.CompilerParams(
                         use_tc_tiling_on_sc=False),
                     scratch_types=scratch)


BR = 1264


def _tc_dense_body(first_layer, last_layer, s0, s1, xa, xb, dinv_a, dinv_b,
                   wo_t, wr_t, b, *outs):
    if first_layer:
        dinv = 1.0 / (dinv_a[...] + dinv_b[...] + 1.0)
    else:
        dinv = dinv_a[...]
    x = jnp.concatenate([xa[...], xb[...]], axis=1)
    agg = jnp.concatenate([s0[...] + xa[...], s1[...] + xb[...]], axis=1)
    agg = agg * dinv[:, None]
    out = (jnp.dot(agg, wo_t[...], preferred_element_type=jnp.float32)
           + jnp.dot(x, wr_t[...], preferred_element_type=jnp.float32)
           + b[...][None, :])
    if last_layer:
        outs[0][...] = out
    else:
        out = jnp.maximum(out, 0.0)
        outs[0][...] = out[:, :DH]
        outs[1][...] = out[:, DH:]
    if first_layer:
        outs[2][...] = dinv


def _tc_dense(first_layer: bool, last_layer: bool):
    if last_layer:
        out_shape = [jax.ShapeDtypeStruct((N_PAD, D), jnp.float32)]
    else:
        out_shape = [jax.ShapeDtypeStruct((N_PAD, DH), jnp.float32),
                     jax.ShapeDtypeStruct((N_PAD, DH), jnp.float32)]
    if first_layer:
        out_shape.append(jax.ShapeDtypeStruct((N_PAD,), jnp.float32))
    return pl.pallas_call(
        functools.partial(_tc_dense_body, first_layer, last_layer),
        out_shape=out_shape,
    )


def kernel(x, edge_index, W_out1, b_out1, W_root1, W_out2, b_out2, W_root2,
           W_out3, b_out3, W_root3):
    row = edge_index[0]
    col = edge_index[1]
    pad = E_PAD - E
    row_p = jnp.concatenate([row, jnp.zeros((pad,), jnp.int32)])
    col_p = jnp.concatenate([col, jnp.full((pad,), TRASH, jnp.int32)])
    row3p = row_p.reshape(NW, NCH_P, CH)
    col3p = col_p.reshape(NW, NCH_P, CH)
    row3a = row_p.reshape(NS, NCH_A, CH)
    x_pad = jnp.zeros((N_PAD, D), jnp.float32).at[:N].set(x)
    xa = x_pad[:, :DH]
    xb = x_pad[:, DH:]

    colp3, d0, d1 = _sc_prep()(row3p, col3p)
    colp3a = colp3.reshape(NS, NCH_A, CH)
    sc = _sc_aggregate()

    s0, s1 = sc(row3a, colp3a, xa, xb)
    ha, hb, dinv = _tc_dense(True, False)(
        s0, s1, xa, xb, d0, d1, W_out1.T, W_root1.T, b_out1)

    s0, s1 = sc(row3a, colp3a, ha, hb)
    ha, hb = _tc_dense(False, False)(
        s0, s1, ha, hb, dinv, dinv, W_out2.T, W_root2.T, b_out2)

    s0, s1 = sc(row3a, colp3a, ha, hb)
    out, = _tc_dense(False, True)(
        s0, s1, ha, hb, dinv, dinv, W_out3.T, W_root3.T, b_out3)

    return out[:N]
